# Initial kernel scaffold; baseline (speedup 1.0000x reference)
import jax
import jax.numpy as jnp
from jax import lax
from jax.experimental import pallas as pl
from jax.experimental.pallas import tpu as pltpu
from jax.experimental.pallas import tpu_sc as plsc

_N = 10000
_D = 128
_V = 10016
_E = 320000
_EL = _E + _N
_NW = 32
_B = 128
_NB = 81
_EPW = _NB * _B
_EPAD = _NW * _EPW
_RPT = _V // 16
_EPS = 1e-5


def _make_sc_layer(heads, with_deg):
    zw = 16 * heads
    mesh = plsc.VectorSubcoreMesh(core_axis_name="c", subcore_axis_name="s")
    n_out = 2 if with_deg else 1
    out_type = [jax.ShapeDtypeStruct((2, _V, 16), jnp.float32)] * n_out
    scratch = [
        pltpu.VMEM((_B,), jnp.int32),
        pltpu.VMEM((_B,), jnp.int32),
        pltpu.VMEM((_B, zw), jnp.float32),
        pltpu.VMEM((_B, 16), jnp.float32),
        pltpu.VMEM((_RPT, 16), jnp.float32),
        pltpu.VMEM_SHARED((_V, 16), jnp.float32),
    ]
    if heads > 1:
        scratch += [
            pltpu.VMEM((_B, 16), jnp.float32),
            pltpu.VMEM((_B, 16), jnp.float32),
            pltpu.VMEM((heads, _B), jnp.float32),
        ]
    if with_deg:
        scratch += [
            pltpu.VMEM((_B, 16), jnp.float32),
            pltpu.VMEM_SHARED((_V, 16), jnp.float32),
        ]

    def body(*refs):
        if heads > 1:
            z_hbm, ps_hbm, pd_hbm, src_hbm, dst_hbm = refs[:5]
            k = 5
        else:
            z_hbm, src_hbm, dst_hbm = refs[:3]
            k = 3
        out_s = refs[k]; k += 1
        if with_deg:
            out_d = refs[k]; k += 1
        sidx, didx, zbuf, mbuf, zrow, acc = refs[k:k + 6]; k += 6
        if heads > 1:
            psbuf, pdbuf, qbuf = refs[k:k + 3]; k += 3
        if with_deg:
            ones, dacc = refs[k:k + 2]

        cid = lax.axis_index("c")
        sid = lax.axis_index("s")
        w = cid * 16 + sid

        zero16 = jnp.zeros((16,), jnp.float32)

        def zero_body(i, _):
            zrow[i, :] = zero16
            return 0
        lax.fori_loop(0, _RPT, zero_body, 0)
        pltpu.sync_copy(zrow, acc.at[pl.ds(sid * _RPT, _RPT)])
        if with_deg:
            pltpu.sync_copy(zrow, dacc.at[pl.ds(sid * _RPT, _RPT)])
            one16 = jnp.ones((16,), jnp.float32)

            def one_body(i, _):
                ones[i, :] = one16
                return 0
            lax.fori_loop(0, _B, one_body, 0)
        plsc.subcore_barrier()

        def batch_body(b, _):
            base = w * _EPW + b * _B
            pltpu.sync_copy(src_hbm.at[pl.ds(base, _B)], sidx)
            pltpu.sync_copy(dst_hbm.at[pl.ds(base, _B)], didx)
            pltpu.sync_copy(z_hbm.at[sidx], zbuf)
            if heads > 1:
                pltpu.sync_copy(ps_hbm.at[sidx], psbuf)
                pltpu.sync_copy(pd_hbm.at[didx], pdbuf)
                for g in range(_B // 16):
                    rows = lax.broadcasted_iota(jnp.int32, (16,), 0) + g * 16
                    t = []
                    for h in range(heads):
                        col = jnp.full((16,), h, jnp.int32)
                        ls = plsc.load_gather(psbuf, [rows, col])
                        ld = plsc.load_gather(pdbuf, [rows, col])
                        t.append(ls - ld)
                    mx = t[0]
                    for h in range(1, heads):
                        mx = jnp.maximum(mx, t[h])
                    ex = [jnp.exp(v - mx) for v in t]
                    tot = ex[0]
                    for h in range(1, heads):
                        tot = tot + ex[h]
                    for h in range(heads):
                        qbuf[h, pl.ds(g * 16, 16)] = ex[h] / tot

                def msg_body(i, _):
                    m = zbuf[i, 0:16] * qbuf[0, i]
                    for h in range(1, heads):
                        m = m + zbuf[i, 16 * h:16 * (h + 1)] * qbuf[h, i]
                    mbuf[i, :] = m
                    return 0
                lax.fori_loop(0, _B, msg_body, 0)
                pltpu.sync_copy(mbuf, acc.at[didx], add=True)
            else:
                pltpu.sync_copy(zbuf, acc.at[didx], add=True)
            if with_deg:
                pltpu.sync_copy(ones, dacc.at[didx], add=True)
            return 0
        lax.fori_loop(0, _NB, batch_body, 0)

        plsc.subcore_barrier()
        sl = pl.ds(sid * _RPT, _RPT)
        pltpu.sync_copy(acc.at[sl], zrow)
        pltpu.sync_copy(zrow, out_s.at[cid].at[sl])
        if with_deg:
            pltpu.sync_copy(dacc.at[sl], zrow)
            pltpu.sync_copy(zrow, out_d.at[cid].at[sl])

    return pl.kernel(body, out_type=out_type, mesh=mesh,
                     scratch_types=scratch)


_sc_feast4_deg = _make_sc_layer(4, True)
_sc_feast4 = _make_sc_layer(4, False)
_sc_feast1 = _make_sc_layer(1, False)


def _rowmask(a):
    rows = lax.broadcasted_iota(jnp.int32, a.shape, 0)
    return jnp.where(rows < _N, a, 0.0)


def _dot(a, b):
    return jnp.dot(a, b, preferred_element_type=jnp.float32)


def _proj(x, w_ref, u_ref, c_ref, z_ref, ps_ref, pd_ref):
    z_ref[...] = _rowmask(_dot(x, w_ref[...]))
    u = u_ref[...]
    up = jnp.zeros((u.shape[0], 16), jnp.float32).at[:, :u.shape[1]].set(u)
    ps = _dot(x, up)
    ps_ref[...] = ps
    c = c_ref[...]
    cp = jnp.zeros((16,), jnp.float32).at[:c.shape[0]].set(c)
    pd_ref[...] = ps - cp[None, :]


def _tc0_body(x_ref, w_ref, u_ref, c_ref, z_ref, ps_ref, pd_ref):
    _proj(x_ref[...], w_ref, u_ref, c_ref, z_ref, ps_ref, pd_ref)


def _tc1_body(s_ref, d_ref, b_ref, w_ref, u_ref, c_ref,
              invc_ref, z_ref, ps_ref, pd_ref):
    invc = 1.0 / jnp.maximum(d_ref[0] + d_ref[1], 1.0)
    invc_ref[...] = invc
    x = jax.nn.relu((s_ref[0] + s_ref[1]) * invc + b_ref[...][None, :])
    x = _rowmask(x)
    _proj(x, w_ref, u_ref, c_ref, z_ref, ps_ref, pd_ref)


def _tc2_body(s_ref, invc_ref, b_ref, w_ref, u_ref, c_ref,
              skip_ref, z_ref, ps_ref, pd_ref):
    t = _rowmask((s_ref[0] + s_ref[1]) * invc_ref[...] + b_ref[...][None, :])
    skip_ref[...] = t
    x = jax.nn.relu(t)
    _proj(x, w_ref, u_ref, c_ref, z_ref, ps_ref, pd_ref)


def _bn(t, g, b):
    tn = t[:_N]
    mu = jnp.mean(tn, axis=0)
    var = jnp.mean((tn - mu[None, :]) ** 2, axis=0)
    return (t - mu[None, :]) / jnp.sqrt(var + _EPS) * g[None, :] + b[None, :]


def _tc3_body(s_ref, invc_ref, b_ref, g_ref, bb_ref, w_ref, z_ref):
    t = (s_ref[0] + s_ref[1]) * invc_ref[...] + b_ref[...][None, :]
    x = jax.nn.relu(_bn(t, g_ref[...], bb_ref[...]))
    z_ref[...] = _rowmask(_dot(x, w_ref[...]))


def _tc4_body(s_ref, invc_ref, b_ref, w_ref, skip_ref, z_ref):
    x = jax.nn.relu((s_ref[0] + s_ref[1]) * invc_ref[...]
                    + b_ref[...][None, :])
    x = _rowmask(x)
    skip_ref[...] = x
    z_ref[...] = _dot(x, w_ref[...])


def _tc5_body(s_ref, invc_ref, b_ref, w_ref, z_ref):
    x = jax.nn.relu((s_ref[0] + s_ref[1]) * invc_ref[...]
                    + b_ref[...][None, :])
    z_ref[...] = _rowmask(_dot(x, w_ref[...]))


def _tc_head_body(s_ref, invc_ref, b_ref, g_ref, bb_ref, skip2_ref,
                  skip4_ref, w1_ref, b1_ref, w2_ref, b2_ref, w3_ref, b3_ref,
                  wo_ref, bo_ref, y_ref):
    t = (s_ref[0] + s_ref[1]) * invc_ref[...] + b_ref[...][None, :]
    x = jax.nn.relu(_bn(t, g_ref[...], bb_ref[...]))
    x = x + skip2_ref[...] + skip4_ref[...]
    x = jax.nn.relu(_dot(x, w1_ref[...]) + b1_ref[...][None, :])
    x = jax.nn.relu(_dot(x, w2_ref[...]) + b2_ref[...][None, :])
    x = jax.nn.relu(_dot(x, w3_ref[...]) + b3_ref[...][None, :])
    x = _dot(x, wo_ref[...]) + bo_ref[...][None, :]
    y_ref[...] = jax.nn.sigmoid(x[:_N])


def _tc(body, out_shape):
    return pl.pallas_call(body, out_shape=out_shape)


_f32 = jnp.float32
_sh = jax.ShapeDtypeStruct
_ZPS = [_sh((_V, 64), _f32), _sh((_V, 16), _f32), _sh((_V, 16), _f32)]
_Z1 = _sh((_V, 16), _f32)


def kernel(x, edge_index, params):
    p = params
    pad = jnp.full((_EPAD - _EL,), _N, jnp.int32)
    loop = jnp.arange(_N, dtype=jnp.int32)
    src = jnp.concatenate([edge_index[0].astype(jnp.int32), loop, pad])
    dst = jnp.concatenate([edge_index[1].astype(jnp.int32), loop, pad])
    xpad = jnp.pad(x, ((0, _V - _N), (0, 0)))

    z, ps, pd = _tc(_tc0_body, _ZPS)(
        xpad, p["conv0_w"], p["conv0_u"], p["conv0_c"])
    s, d = _sc_feast4_deg(z, ps, pd, src, dst)

    invc, z, ps, pd = _tc(_tc1_body, [_Z1] + _ZPS)(
        s, d, p["conv0_b"], p["conv1_w"], p["conv1_u"], p["conv1_c"])
    (s,) = _sc_feast4(z, ps, pd, src, dst)

    skip2, z, ps, pd = _tc(_tc2_body, [_Z1] + _ZPS)(
        s, invc, p["conv1_b"], p["conv2_w"], p["conv2_u"], p["conv2_c"])
    (s,) = _sc_feast4(z, ps, pd, src, dst)

    z = _tc(_tc3_body, _Z1)(
        s, invc, p["conv2_b"], p["bn1_g"], p["bn1_b"], p["conv3_w"])
    (s,) = _sc_feast1(z, src, dst)

    skip4, z = _tc(_tc4_body, [_Z1, _Z1])(
        s, invc, p["conv3_b"], p["conv4_w"])
    (s,) = _sc_feast1(z, src, dst)

    z = _tc(_tc5_body, _Z1)(s, invc, p["conv4_b"], p["conv5_w"])
    (s,) = _sc_feast1(z, src, dst)

    y = _tc(_tc_head_body, _sh((_N, 1), _f32))(
        s, invc, p["conv5_b"], p["bn2_g"], p["bn2_b"], skip2, skip4,
        p["lin1_w"], p["lin1_b"], p["lin2_w"], p["lin2_b"],
        p["lin3_w"], p["lin3_b"], p["out_w"], p["out_b"])
    return y


# speedup vs baseline: 6.4524x; 6.4524x over previous
import jax
import jax.numpy as jnp
from jax import lax
from jax.experimental import pallas as pl
from jax.experimental.pallas import tpu as pltpu
from jax.experimental.pallas import tpu_sc as plsc

_N = 10000
_D = 128
_V = 10112
_E = 320000
_EL = _E + _N
_NW = 32
_B = 128
_NB = 81
_EPW = _NB * _B
_EPAD = _NW * _EPW
_RPT = _V // 16
_EPS = 1e-5


def _make_sc_layer(heads, with_deg):
    zw = 16 * heads
    mesh = plsc.VectorSubcoreMesh(core_axis_name="c", subcore_axis_name="s")
    n_out = 2 if with_deg else 1
    out_type = [jax.ShapeDtypeStruct((2, _V, 16), jnp.float32)] * n_out
    scratch = [
        pltpu.VMEM((_B,), jnp.int32),
        pltpu.VMEM((_B,), jnp.int32),
        pltpu.VMEM((_B, zw), jnp.float32),
        pltpu.VMEM((_B, 16), jnp.float32),
        pltpu.VMEM((_RPT, 16), jnp.float32),
        pltpu.VMEM_SHARED((_V, 16), jnp.float32),
    ]
    if heads > 1:
        scratch += [
            pltpu.VMEM((_B, 16), jnp.float32),
            pltpu.VMEM((_B, 16), jnp.float32),
        ]
    if with_deg:
        scratch += [
            pltpu.VMEM((_B, 16), jnp.float32),
            pltpu.VMEM_SHARED((_V, 16), jnp.float32),
        ]

    def body(*refs):
        if heads > 1:
            z_hbm, ps_hbm, pd_hbm, src_hbm, dst_hbm = refs[:5]
            k = 5
        else:
            z_hbm, src_hbm, dst_hbm = refs[:3]
            k = 3
        out_s = refs[k]; k += 1
        if with_deg:
            out_d = refs[k]; k += 1
        sidx, didx, zbuf, mbuf, zrow, acc = refs[k:k + 6]; k += 6
        if heads > 1:
            psbuf, pdbuf = refs[k:k + 2]; k += 2
        if with_deg:
            ones, dacc = refs[k:k + 2]

        cid = lax.axis_index("c")
        sid = lax.axis_index("s")
        w = cid * 16 + sid

        zero16 = jnp.zeros((16,), jnp.float32)

        def zero_body(i, _):
            zrow[i, :] = zero16
            return 0
        lax.fori_loop(0, _RPT, zero_body, 0)
        pltpu.sync_copy(zrow, acc.at[pl.ds(sid * _RPT, _RPT)])
        if with_deg:
            pltpu.sync_copy(zrow, dacc.at[pl.ds(sid * _RPT, _RPT)])
            one16 = jnp.ones((16,), jnp.float32)

            def one_body(i, _):
                ones[i, :] = one16
                return 0
            lax.fori_loop(0, _B, one_body, 0)
        plsc.subcore_barrier()

        def batch_body(b, _):
            base = w * _EPW + b * _B
            pltpu.sync_copy(src_hbm.at[pl.ds(base, _B)], sidx)
            pltpu.sync_copy(dst_hbm.at[pl.ds(base, _B)], didx)
            pltpu.sync_copy(z_hbm.at[sidx], zbuf)
            if heads > 1:
                pltpu.sync_copy(ps_hbm.at[sidx], psbuf)
                pltpu.sync_copy(pd_hbm.at[didx], pdbuf)
                for g in range(_B // 16):
                    rows = lax.broadcasted_iota(jnp.int32, (16,), 0) + g * 16
                    t = []
                    for h in range(heads):
                        col = jnp.full((16,), h, jnp.int32)
                        ls = plsc.load_gather(psbuf, [rows, col])
                        ld = plsc.load_gather(pdbuf, [rows, col])
                        t.append(ls - ld)
                    mx = t[0]
                    for h in range(1, heads):
                        mx = jnp.maximum(mx, t[h])
                    ex = [jnp.exp(v - mx) for v in t]
                    tot = ex[0]
                    for h in range(1, heads):
                        tot = tot + ex[h]
                    q = [e / tot for e in ex]
                    for c in range(16):
                        col = jnp.full((16,), c, jnp.int32)
                        m = plsc.load_gather(zbuf, [rows, col]) * q[0]
                        for h in range(1, heads):
                            colh = jnp.full((16,), 16 * h + c, jnp.int32)
                            m = m + plsc.load_gather(zbuf, [rows, colh]) * q[h]
                        plsc.store_scatter(mbuf, [rows, col], m)
                pltpu.sync_copy(mbuf, acc.at[didx], add=True)
            else:
                pltpu.sync_copy(zbuf, acc.at[didx], add=True)
            if with_deg:
                pltpu.sync_copy(ones, dacc.at[didx], add=True)
            return 0
        lax.fori_loop(0, _NB, batch_body, 0)

        plsc.subcore_barrier()
        sl = pl.ds(sid * _RPT, _RPT)
        pltpu.sync_copy(acc.at[sl], zrow)
        pltpu.sync_copy(zrow, out_s.at[cid].at[sl])
        if with_deg:
            pltpu.sync_copy(dacc.at[sl], zrow)
            pltpu.sync_copy(zrow, out_d.at[cid].at[sl])

    return pl.kernel(body, out_type=out_type, mesh=mesh,
                     scratch_types=scratch,
                     compiler_params=pltpu.CompilerParams(
                         needs_layout_passes=False,
                         use_tc_tiling_on_sc=False))


_sc_feast4_deg = _make_sc_layer(4, True)
_sc_feast4 = _make_sc_layer(4, False)
_sc_feast1 = _make_sc_layer(1, False)


def _rowmask(a):
    rows = lax.broadcasted_iota(jnp.int32, a.shape, 0)
    return jnp.where(rows < _N, a, 0.0)


def _dot(a, b):
    return jnp.dot(a, b, preferred_element_type=jnp.float32)


def _proj(x, w_ref, u_ref, c_ref, z_ref, ps_ref, pd_ref):
    z_ref[...] = _rowmask(_dot(x, w_ref[...]))
    u = u_ref[...]
    up = jnp.pad(u, ((0, 0), (0, 16 - u.shape[1])))
    ps = _dot(x, up)
    ps_ref[...] = ps
    c = c_ref[...]
    cp = jnp.pad(c, (0, 16 - c.shape[0]))
    pd_ref[...] = ps - cp[None, :]


def _tc0_body(x_ref, w_ref, u_ref, c_ref, z_ref, ps_ref, pd_ref):
    _proj(x_ref[...], w_ref, u_ref, c_ref, z_ref, ps_ref, pd_ref)


def _tc1_body(s_ref, d_ref, b_ref, w_ref, u_ref, c_ref,
              invc_ref, z_ref, ps_ref, pd_ref):
    invc = 1.0 / jnp.maximum(d_ref[0] + d_ref[1], 1.0)
    invc_ref[...] = invc
    x = jax.nn.relu((s_ref[0] + s_ref[1]) * invc + b_ref[...][None, :])
    x = _rowmask(x)
    _proj(x, w_ref, u_ref, c_ref, z_ref, ps_ref, pd_ref)


def _tc2_body(s_ref, invc_ref, b_ref, w_ref, u_ref, c_ref,
              skip_ref, z_ref, ps_ref, pd_ref):
    t = _rowmask((s_ref[0] + s_ref[1]) * invc_ref[...] + b_ref[...][None, :])
    skip_ref[...] = t
    x = jax.nn.relu(t)
    _proj(x, w_ref, u_ref, c_ref, z_ref, ps_ref, pd_ref)


def _bn(t, g, b):
    tn = t[:_N]
    mu = jnp.mean(tn, axis=0)
    var = jnp.mean((tn - mu[None, :]) ** 2, axis=0)
    return (t - mu[None, :]) / jnp.sqrt(var + _EPS) * g[None, :] + b[None, :]


def _tc3_body(s_ref, invc_ref, b_ref, g_ref, bb_ref, w_ref, z_ref):
    t = (s_ref[0] + s_ref[1]) * invc_ref[...] + b_ref[...][None, :]
    x = jax.nn.relu(_bn(t, g_ref[...], bb_ref[...]))
    z_ref[...] = _rowmask(_dot(x, w_ref[...]))


def _tc4_body(s_ref, invc_ref, b_ref, w_ref, skip_ref, z_ref):
    x = jax.nn.relu((s_ref[0] + s_ref[1]) * invc_ref[...]
                    + b_ref[...][None, :])
    x = _rowmask(x)
    skip_ref[...] = x
    z_ref[...] = _dot(x, w_ref[...])


def _tc5_body(s_ref, invc_ref, b_ref, w_ref, z_ref):
    x = jax.nn.relu((s_ref[0] + s_ref[1]) * invc_ref[...]
                    + b_ref[...][None, :])
    z_ref[...] = _rowmask(_dot(x, w_ref[...]))


def _tc_head_body(s_ref, invc_ref, b_ref, g_ref, bb_ref, skip2_ref,
                  skip4_ref, w1_ref, b1_ref, w2_ref, b2_ref, w3_ref, b3_ref,
                  wo_ref, bo_ref, y_ref):
    t = (s_ref[0] + s_ref[1]) * invc_ref[...] + b_ref[...][None, :]
    x = jax.nn.relu(_bn(t, g_ref[...], bb_ref[...]))
    x = x + skip2_ref[...] + skip4_ref[...]
    x = jax.nn.relu(_dot(x, w1_ref[...]) + b1_ref[...][None, :])
    x = jax.nn.relu(_dot(x, w2_ref[...]) + b2_ref[...][None, :])
    x = jax.nn.relu(_dot(x, w3_ref[...]) + b3_ref[...][None, :])
    x = _dot(x, wo_ref[...]) + bo_ref[...][None, :]
    y_ref[...] = jax.nn.sigmoid(x[:_N])


def _tc(body, out_shape):
    return pl.pallas_call(body, out_shape=out_shape)


_f32 = jnp.float32
_sh = jax.ShapeDtypeStruct
_ZPS = [_sh((_V, 64), _f32), _sh((_V, 16), _f32), _sh((_V, 16), _f32)]
_Z1 = _sh((_V, 16), _f32)


def kernel(x, edge_index, params):
    p = params
    pad = jnp.full((_EPAD - _EL,), _N, jnp.int32)
    loop = jnp.arange(_N, dtype=jnp.int32)
    src = jnp.concatenate([edge_index[0].astype(jnp.int32), loop, pad])
    dst = jnp.concatenate([edge_index[1].astype(jnp.int32), loop, pad])
    xpad = jnp.pad(x, ((0, _V - _N), (0, 0)))

    z, ps, pd = _tc(_tc0_body, _ZPS)(
        xpad, p["conv0_w"], p["conv0_u"], p["conv0_c"])
    s, d = _sc_feast4_deg(z, ps, pd, src, dst)

    invc, z, ps, pd = _tc(_tc1_body, [_Z1] + _ZPS)(
        s, d, p["conv0_b"], p["conv1_w"], p["conv1_u"], p["conv1_c"])
    (s,) = _sc_feast4(z, ps, pd, src, dst)

    skip2, z, ps, pd = _tc(_tc2_body, [_Z1] + _ZPS)(
        s, invc, p["conv1_b"], p["conv2_w"], p["conv2_u"], p["conv2_c"])
    (s,) = _sc_feast4(z, ps, pd, src, dst)

    z = _tc(_tc3_body, _Z1)(
        s, invc, p["conv2_b"], p["bn1_g"], p["bn1_b"], p["conv3_w"])
    (s,) = _sc_feast1(z, src, dst)

    skip4, z = _tc(_tc4_body, [_Z1, _Z1])(
        s, invc, p["conv3_b"], p["conv4_w"])
    (s,) = _sc_feast1(z, src, dst)

    z = _tc(_tc5_body, _Z1)(s, invc, p["conv4_b"], p["conv5_w"])
    (s,) = _sc_feast1(z, src, dst)

    y = _tc(_tc_head_body, _sh((_N, 1), _f32))(
        s, invc, p["conv5_b"], p["bn2_g"], p["bn2_b"], skip2, skip4,
        p["lin1_w"], p["lin1_b"], p["lin2_w"], p["lin2_b"],
        p["lin3_w"], p["lin3_b"], p["out_w"], p["out_b"])
    return y


# speedup vs baseline: 6.5764x; 1.0192x over previous
import jax
import jax.numpy as jnp
from jax import lax
from jax.experimental import pallas as pl
from jax.experimental.pallas import tpu as pltpu
from jax.experimental.pallas import tpu_sc as plsc

_N = 10000
_D = 128
_V = 10112
_E = 320000
_EL = _E + _N
_NW = 32
_B = 128
_NB = 82
_EPW = _NB * _B
_EPAD = _NW * _EPW
_RPT = _V // 16
_EPS = 1e-5


def _make_sc_layer(heads, with_deg, pipelined=True):
    zw = 16 * heads
    mesh = plsc.VectorSubcoreMesh(core_axis_name="c", subcore_axis_name="s")
    n_out = 2 if with_deg else 1
    out_type = [jax.ShapeDtypeStruct((2, _V, 16), jnp.float32)] * n_out
    dma = pltpu.SemaphoreType.DMA
    scratch = [
        pltpu.VMEM((_NB + 1, _B), jnp.int32),
        pltpu.VMEM((_NB + 1, _B), jnp.int32),
        pltpu.VMEM((_B, zw), jnp.float32),
        pltpu.VMEM((_B, zw), jnp.float32),
        pltpu.VMEM((_RPT, 16), jnp.float32),
        pltpu.VMEM_SHARED((_V, 16), jnp.float32),
        dma, dma,
    ]
    if heads > 1:
        scratch += [
            pltpu.VMEM((_B, 16), jnp.float32),
            pltpu.VMEM((_B, 16), jnp.float32),
            pltpu.VMEM((_B, 16), jnp.float32),
            pltpu.VMEM((_B, 16), jnp.float32),
            pltpu.VMEM((_B, 16), jnp.float32),
            pltpu.VMEM((heads, _B), jnp.float32),
            dma, dma, dma, dma,
        ]
    if with_deg:
        scratch += [
            pltpu.VMEM((_B, 16), jnp.float32),
            pltpu.VMEM_SHARED((_V, 16), jnp.float32),
        ]

    def body(*refs):
        if heads > 1:
            z_hbm, ps_hbm, pd_hbm, src_hbm, dst_hbm = refs[:5]
            k = 5
        else:
            z_hbm, src_hbm, dst_hbm = refs[:3]
            k = 3
        out_s = refs[k]; k += 1
        if with_deg:
            out_d = refs[k]; k += 1
        sidx, didx, z0, z1, zrow, acc, gz0, gz1 = refs[k:k + 8]
        k += 8
        zb = (z0, z1)
        gz = (gz0, gz1)
        if heads > 1:
            ps0, ps1, pd0, pd1, mbuf, qbuf, gps0, gps1, gpd0, gpd1 = \
                refs[k:k + 10]
            k += 10
            psb, pdb = (ps0, ps1), (pd0, pd1)
            gps, gpd = (gps0, gps1), (gpd0, gpd1)
        if with_deg:
            ones, dacc = refs[k:k + 2]

        cid = lax.axis_index("c")
        sid = lax.axis_index("s")
        w = cid * 16 + sid

        zero16 = jnp.zeros((16,), jnp.float32)

        def zero_body(i, _):
            zrow[i, :] = zero16
            return 0
        lax.fori_loop(0, _RPT, zero_body, 0)
        pltpu.sync_copy(zrow, acc.at[pl.ds(sid * _RPT, _RPT)])
        if with_deg:
            pltpu.sync_copy(zrow, dacc.at[pl.ds(sid * _RPT, _RPT)])
            one16 = jnp.ones((16,), jnp.float32)

            def one_body(i, _):
                ones[i, :] = one16
                return 0
            lax.fori_loop(0, _B, one_body, 0)
        pltpu.sync_copy(src_hbm.at[w], sidx.at[pl.ds(0, _NB)])
        pltpu.sync_copy(dst_hbm.at[w], didx.at[pl.ds(0, _NB)])
        zero16i = jnp.zeros((16,), jnp.int32)
        for cc in range(_B // 16):
            sidx[_NB, pl.ds(16 * cc, 16)] = zero16i
        plsc.subcore_barrier()

        def gathers(b, slot):
            ds = [pltpu.make_async_copy(z_hbm.at[sidx.at[b]], zb[slot],
                                        gz[slot])]
            if heads > 1:
                ds.append(pltpu.make_async_copy(ps_hbm.at[sidx.at[b]],
                                                psb[slot], gps[slot]))
                ds.append(pltpu.make_async_copy(pd_hbm.at[didx.at[b]],
                                                pdb[slot], gpd[slot]))
            return ds


        def step(b, slot, prefetch=True):
            if heads > 1:
                gathers(b, slot)[0].wait()
                if prefetch:
                    gathers(b + 1, 1 - slot)[0].start()
                psbuf, pdbuf = psb[slot], pdb[slot]
                pltpu.sync_copy(ps_hbm.at[sidx.at[b]], psbuf)
                pltpu.sync_copy(pd_hbm.at[didx.at[b]], pdbuf)
                for g in range(_B // 16):
                    rows = lax.broadcasted_iota(jnp.int32, (16,), 0) + g * 16
                    t = []
                    for h in range(heads):
                        col = jnp.full((16,), h, jnp.int32)
                        ls = plsc.load_gather(psbuf, [rows, col])
                        ld = plsc.load_gather(pdbuf, [rows, col])
                        t.append(ls - ld)
                    mx = t[0]
                    for h in range(1, heads):
                        mx = jnp.maximum(mx, t[h])
                    ex = [jnp.exp(v - mx) for v in t]
                    tot = ex[0]
                    for h in range(1, heads):
                        tot = tot + ex[h]
                    for h in range(heads):
                        qbuf[h, pl.ds(g * 16, 16)] = ex[h] / tot
                zbuf = zb[slot]
                for g in range(_B // 16):
                    rows = lax.broadcasted_iota(jnp.int32, (16,), 0) + g * 16
                    q = [qbuf[h, pl.ds(g * 16, 16)] for h in range(heads)]
                    for c in range(16):
                        col = jnp.full((16,), c, jnp.int32)
                        m = plsc.load_gather(zbuf, [rows, col]) * q[0]
                        for h in range(1, heads):
                            colh = jnp.full((16,), 16 * h + c, jnp.int32)
                            m = m + plsc.load_gather(zbuf, [rows, colh]) * q[h]
                        plsc.store_scatter(mbuf, [rows, col], m)
                pltpu.sync_copy(mbuf, acc.at[didx.at[b]], add=True)
            else:
                gathers(b, slot)[0].wait()
                if prefetch:
                    for d in gathers(b + 1, 1 - slot):
                        d.start()
                pltpu.sync_copy(zb[slot], acc.at[didx.at[b]], add=True)
            if with_deg:
                pltpu.sync_copy(ones, dacc.at[didx.at[b]], add=True)

        if pipelined:
            gathers(0, 0)[0].start()

            def pair_body(g, _):
                for slot in (0, 1):
                    step(2 * g + slot, slot)
                return 0
            lax.fori_loop(0, _NB // 2, pair_body, 0)

            gathers(_NB, 0)[0].wait()
        else:
            def batch_body(b, _):
                gathers(b, 0)[0].start()
                step(b, 0, prefetch=False)
                return 0
            lax.fori_loop(0, _NB, batch_body, 0)

        plsc.subcore_barrier()
        sl = pl.ds(sid * _RPT, _RPT)
        pltpu.sync_copy(acc.at[sl], zrow)
        pltpu.sync_copy(zrow, out_s.at[cid].at[sl])
        if with_deg:
            pltpu.sync_copy(dacc.at[sl], zrow)
            pltpu.sync_copy(zrow, out_d.at[cid].at[sl])

    return pl.kernel(body, out_type=out_type, mesh=mesh,
                     scratch_types=scratch,
                     compiler_params=pltpu.CompilerParams(
                         needs_layout_passes=False,
                         use_tc_tiling_on_sc=False))


_sc_feast4_deg = _make_sc_layer(4, True)
_sc_feast4 = _make_sc_layer(4, False)
_sc_feast1 = _make_sc_layer(1, False)


def _rowmask(a):
    rows = lax.broadcasted_iota(jnp.int32, a.shape, 0)
    return jnp.where(rows < _N, a, 0.0)


def _dot(a, b):
    return jnp.dot(a, b, preferred_element_type=jnp.float32)


def _proj(x, w_ref, u_ref, c_ref, z_ref, ps_ref, pd_ref):
    z_ref[...] = _rowmask(_dot(x, w_ref[...]))
    u = u_ref[...]
    up = jnp.pad(u, ((0, 0), (0, 16 - u.shape[1])))
    ps = _dot(x, up)
    ps_ref[...] = ps
    c = c_ref[...]
    cp = jnp.pad(c, (0, 16 - c.shape[0]))
    pd_ref[...] = ps - cp[None, :]


def _tc0_body(x_ref, w_ref, u_ref, c_ref, z_ref, ps_ref, pd_ref):
    _proj(x_ref[...], w_ref, u_ref, c_ref, z_ref, ps_ref, pd_ref)


def _tc1_body(s_ref, d_ref, b_ref, w_ref, u_ref, c_ref,
              invc_ref, z_ref, ps_ref, pd_ref):
    invc = 1.0 / jnp.maximum(d_ref[0] + d_ref[1], 1.0)
    invc_ref[...] = invc
    x = jax.nn.relu((s_ref[0] + s_ref[1]) * invc + b_ref[...][None, :])
    x = _rowmask(x)
    _proj(x, w_ref, u_ref, c_ref, z_ref, ps_ref, pd_ref)


def _tc2_body(s_ref, invc_ref, b_ref, w_ref, u_ref, c_ref,
              skip_ref, z_ref, ps_ref, pd_ref):
    t = _rowmask((s_ref[0] + s_ref[1]) * invc_ref[...] + b_ref[...][None, :])
    skip_ref[...] = t
    x = jax.nn.relu(t)
    _proj(x, w_ref, u_ref, c_ref, z_ref, ps_ref, pd_ref)


def _bn(t, g, b):
    tn = t[:_N]
    mu = jnp.mean(tn, axis=0)
    var = jnp.mean((tn - mu[None, :]) ** 2, axis=0)
    return (t - mu[None, :]) / jnp.sqrt(var + _EPS) * g[None, :] + b[None, :]


def _tc3_body(s_ref, invc_ref, b_ref, g_ref, bb_ref, w_ref, z_ref):
    t = (s_ref[0] + s_ref[1]) * invc_ref[...] + b_ref[...][None, :]
    x = jax.nn.relu(_bn(t, g_ref[...], bb_ref[...]))
    z_ref[...] = _rowmask(_dot(x, w_ref[...]))


def _tc4_body(s_ref, invc_ref, b_ref, w_ref, skip_ref, z_ref):
    x = jax.nn.relu((s_ref[0] + s_ref[1]) * invc_ref[...]
                    + b_ref[...][None, :])
    x = _rowmask(x)
    skip_ref[...] = x
    z_ref[...] = _dot(x, w_ref[...])


def _tc5_body(s_ref, invc_ref, b_ref, w_ref, z_ref):
    x = jax.nn.relu((s_ref[0] + s_ref[1]) * invc_ref[...]
                    + b_ref[...][None, :])
    z_ref[...] = _rowmask(_dot(x, w_ref[...]))


def _tc_head_body(s_ref, invc_ref, b_ref, g_ref, bb_ref, skip2_ref,
                  skip4_ref, w1_ref, b1_ref, w2_ref, b2_ref, w3_ref, b3_ref,
                  wo_ref, bo_ref, y_ref):
    t = (s_ref[0] + s_ref[1]) * invc_ref[...] + b_ref[...][None, :]
    x = jax.nn.relu(_bn(t, g_ref[...], bb_ref[...]))
    x = x + skip2_ref[...] + skip4_ref[...]
    x = jax.nn.relu(_dot(x, w1_ref[...]) + b1_ref[...][None, :])
    x = jax.nn.relu(_dot(x, w2_ref[...]) + b2_ref[...][None, :])
    x = jax.nn.relu(_dot(x, w3_ref[...]) + b3_ref[...][None, :])
    x = _dot(x, wo_ref[...]) + bo_ref[...][None, :]
    y_ref[...] = jax.nn.sigmoid(x[:_N])


def _tc(body, out_shape):
    return pl.pallas_call(body, out_shape=out_shape)


_f32 = jnp.float32
_sh = jax.ShapeDtypeStruct
_ZPS = [_sh((_V, 64), _f32), _sh((_V, 16), _f32), _sh((_V, 16), _f32)]
_Z1 = _sh((_V, 16), _f32)


def kernel(x, edge_index, params):
    p = params
    pad = jnp.full((_EPAD - _EL,), _N, jnp.int32)
    loop = jnp.arange(_N, dtype=jnp.int32)
    src = jnp.concatenate([edge_index[0].astype(jnp.int32), loop, pad])
    src = src.reshape(_NW, _NB, _B)
    dst = jnp.concatenate([edge_index[1].astype(jnp.int32), loop, pad])
    dst = dst.reshape(_NW, _NB, _B)
    xpad = jnp.pad(x, ((0, _V - _N), (0, 0)))

    z, ps, pd = _tc(_tc0_body, _ZPS)(
        xpad, p["conv0_w"], p["conv0_u"], p["conv0_c"])
    s, d = _sc_feast4_deg(z, ps, pd, src, dst)

    invc, z, ps, pd = _tc(_tc1_body, [_Z1] + _ZPS)(
        s, d, p["conv0_b"], p["conv1_w"], p["conv1_u"], p["conv1_c"])
    (s,) = _sc_feast4(z, ps, pd, src, dst)

    skip2, z, ps, pd = _tc(_tc2_body, [_Z1] + _ZPS)(
        s, invc, p["conv1_b"], p["conv2_w"], p["conv2_u"], p["conv2_c"])
    (s,) = _sc_feast4(z, ps, pd, src, dst)

    z = _tc(_tc3_body, _Z1)(
        s, invc, p["conv2_b"], p["bn1_g"], p["bn1_b"], p["conv3_w"])
    (s,) = _sc_feast1(z, src, dst)

    skip4, z = _tc(_tc4_body, [_Z1, _Z1])(
        s, invc, p["conv3_b"], p["conv4_w"])
    (s,) = _sc_feast1(z, src, dst)

    z = _tc(_tc5_body, _Z1)(s, invc, p["conv4_b"], p["conv5_w"])
    (s,) = _sc_feast1(z, src, dst)

    y = _tc(_tc_head_body, _sh((_N, 1), _f32))(
        s, invc, p["conv5_b"], p["bn2_g"], p["bn2_b"], skip2, skip4,
        p["lin1_w"], p["lin1_b"], p["lin2_w"], p["lin2_b"],
        p["lin3_w"], p["lin3_b"], p["out_w"], p["out_b"])
    return y


# speedup vs baseline: 8.4082x; 1.2786x over previous
import jax
import jax.numpy as jnp
from jax import lax
from jax.experimental import pallas as pl
from jax.experimental.pallas import tpu as pltpu
from jax.experimental.pallas import tpu_sc as plsc

_N = 10000
_D = 128
_V = 10112
_E = 320000
_EL = _E + _N
_NW = 32
_B = 128
_NB = 82
_EPW = _NB * _B
_EPAD = _NW * _EPW
_RPT = _V // 16
_EPS = 1e-5


def _make_sc_layer(heads, with_deg, pipelined=True):
    zw = 16 * heads + (16 if heads > 1 else 0)
    mesh = plsc.VectorSubcoreMesh(core_axis_name="c", subcore_axis_name="s")
    n_out = 2 if with_deg else 1
    out_type = [jax.ShapeDtypeStruct((2, _V, 16), jnp.float32)] * n_out
    dma = pltpu.SemaphoreType.DMA
    scratch = [
        pltpu.VMEM((_NB + 1, _B), jnp.int32),
        pltpu.VMEM((_NB + 1, _B), jnp.int32),
        pltpu.VMEM((_B, zw), jnp.float32),
        pltpu.VMEM((_B, zw), jnp.float32),
        pltpu.VMEM((_RPT, 16), jnp.float32),
        pltpu.VMEM_SHARED((_V, 16), jnp.float32),
        dma, dma,
    ]
    if heads > 1:
        scratch += [
            pltpu.VMEM((_B, 16), jnp.float32),
            pltpu.VMEM((_B, 16), jnp.float32),
        ]
    if with_deg:
        scratch += [
            pltpu.VMEM((_B, 16), jnp.float32),
            pltpu.VMEM_SHARED((_V, 16), jnp.float32),
        ]

    def body(*refs):
        if heads > 1:
            z_hbm, pd_hbm, src_hbm, dst_hbm = refs[:4]
            k = 4
        else:
            z_hbm, src_hbm, dst_hbm = refs[:3]
            k = 3
        out_s = refs[k]; k += 1
        if with_deg:
            out_d = refs[k]; k += 1
        sidx, didx, z0, z1, zrow, acc, gz0, gz1 = refs[k:k + 8]
        k += 8
        zb = (z0, z1)
        gz = (gz0, gz1)
        if heads > 1:
            pdbuf, mbuf = refs[k:k + 2]
            k += 2
        if with_deg:
            ones, dacc = refs[k:k + 2]

        cid = lax.axis_index("c")
        sid = lax.axis_index("s")
        w = cid * 16 + sid

        zero16 = jnp.zeros((16,), jnp.float32)

        def zero_body(i, _):
            zrow[i, :] = zero16
            return 0
        lax.fori_loop(0, _RPT, zero_body, 0)
        pltpu.sync_copy(zrow, acc.at[pl.ds(sid * _RPT, _RPT)])
        if with_deg:
            pltpu.sync_copy(zrow, dacc.at[pl.ds(sid * _RPT, _RPT)])
            one16 = jnp.ones((16,), jnp.float32)

            def one_body(i, _):
                ones[i, :] = one16
                return 0
            lax.fori_loop(0, _B, one_body, 0)
        pltpu.sync_copy(src_hbm.at[w], sidx.at[pl.ds(0, _NB)])
        pltpu.sync_copy(dst_hbm.at[w], didx.at[pl.ds(0, _NB)])
        zero16i = jnp.zeros((16,), jnp.int32)
        for cc in range(_B // 16):
            sidx[_NB, pl.ds(16 * cc, 16)] = zero16i
        plsc.subcore_barrier()

        def gathers(b, slot):
            return [pltpu.make_async_copy(z_hbm.at[sidx.at[b]], zb[slot],
                                          gz[slot])]

        def step(b, slot, prefetch=True):
            if heads > 1:
                gathers(b, slot)[0].wait()
                if prefetch:
                    gathers(b + 1, 1 - slot)[0].start()
                pltpu.sync_copy(pd_hbm.at[didx.at[b]], pdbuf)
                zbuf = zb[slot]

                def edge(e):
                    t = zbuf[e, pl.ds(16 * heads, 16)] - pdbuf[e, :]
                    ev = jnp.exp(t)
                    s = jnp.sum(ev) - (16.0 - heads)
                    m = zbuf[e, 0:16] * ev[0]
                    for h in range(1, heads):
                        m = m + zbuf[e, pl.ds(16 * h, 16)] * ev[h]
                    mbuf[e, :] = m / jnp.broadcast_to(s, (16,))

                def egroup(g, _):
                    for j in range(8):
                        edge(g * 8 + j)
                    return 0
                lax.fori_loop(0, _B // 8, egroup, 0)
                pltpu.sync_copy(mbuf, acc.at[didx.at[b]], add=True)
            else:
                gathers(b, slot)[0].wait()
                if prefetch:
                    for d in gathers(b + 1, 1 - slot):
                        d.start()
                pltpu.sync_copy(zb[slot], acc.at[didx.at[b]], add=True)
            if with_deg:
                pltpu.sync_copy(ones, dacc.at[didx.at[b]], add=True)

        if pipelined:
            gathers(0, 0)[0].start()

            def pair_body(g, _):
                for slot in (0, 1):
                    step(2 * g + slot, slot)
                return 0
            lax.fori_loop(0, _NB // 2, pair_body, 0)

            gathers(_NB, 0)[0].wait()
        else:
            def batch_body(b, _):
                gathers(b, 0)[0].start()
                step(b, 0, prefetch=False)
                return 0
            lax.fori_loop(0, _NB, batch_body, 0)

        plsc.subcore_barrier()
        sl = pl.ds(sid * _RPT, _RPT)
        pltpu.sync_copy(acc.at[sl], zrow)
        pltpu.sync_copy(zrow, out_s.at[cid].at[sl])
        if with_deg:
            pltpu.sync_copy(dacc.at[sl], zrow)
            pltpu.sync_copy(zrow, out_d.at[cid].at[sl])

    return pl.kernel(body, out_type=out_type, mesh=mesh,
                     scratch_types=scratch,
                     compiler_params=pltpu.CompilerParams(
                         needs_layout_passes=False,
                         use_tc_tiling_on_sc=False))


_sc_feast4_deg = _make_sc_layer(4, True)
_sc_feast4 = _make_sc_layer(4, False)
_sc_feast1 = _make_sc_layer(1, False)


def _rowmask(a):
    rows = lax.broadcasted_iota(jnp.int32, a.shape, 0)
    return jnp.where(rows < _N, a, 0.0)


def _dot(a, b):
    return jnp.dot(a, b, preferred_element_type=jnp.float32)


def _proj(x, w_ref, u_ref, c_ref, zps_ref, pd_ref):
    z = _rowmask(_dot(x, w_ref[...]))
    u = u_ref[...]
    up = jnp.pad(u, ((0, 0), (0, 16 - u.shape[1])))
    ps = _dot(x, up)
    zps_ref[...] = jnp.concatenate([z, ps], axis=1)
    c = c_ref[...]
    cp = jnp.pad(c, (0, 16 - c.shape[0]))
    pd_ref[...] = ps - cp[None, :]


def _tc0_body(x_ref, w_ref, u_ref, c_ref, zps_ref, pd_ref):
    _proj(x_ref[...], w_ref, u_ref, c_ref, zps_ref, pd_ref)


def _tc1_body(s_ref, d_ref, b_ref, w_ref, u_ref, c_ref,
              invc_ref, zps_ref, pd_ref):
    invc = 1.0 / jnp.maximum(d_ref[0] + d_ref[1], 1.0)
    invc_ref[...] = invc
    x = jax.nn.relu((s_ref[0] + s_ref[1]) * invc + b_ref[...][None, :])
    x = _rowmask(x)
    _proj(x, w_ref, u_ref, c_ref, zps_ref, pd_ref)


def _tc2_body(s_ref, invc_ref, b_ref, w_ref, u_ref, c_ref,
              skip_ref, zps_ref, pd_ref):
    t = _rowmask((s_ref[0] + s_ref[1]) * invc_ref[...] + b_ref[...][None, :])
    skip_ref[...] = t
    x = jax.nn.relu(t)
    _proj(x, w_ref, u_ref, c_ref, zps_ref, pd_ref)


def _bn(t, g, b):
    tn = t[:_N]
    mu = jnp.mean(tn, axis=0)
    var = jnp.mean((tn - mu[None, :]) ** 2, axis=0)
    return (t - mu[None, :]) / jnp.sqrt(var + _EPS) * g[None, :] + b[None, :]


def _tc3_body(s_ref, invc_ref, b_ref, g_ref, bb_ref, w_ref, z_ref):
    t = (s_ref[0] + s_ref[1]) * invc_ref[...] + b_ref[...][None, :]
    x = jax.nn.relu(_bn(t, g_ref[...], bb_ref[...]))
    z_ref[...] = _rowmask(_dot(x, w_ref[...]))


def _tc4_body(s_ref, invc_ref, b_ref, w_ref, skip_ref, z_ref):
    x = jax.nn.relu((s_ref[0] + s_ref[1]) * invc_ref[...]
                    + b_ref[...][None, :])
    x = _rowmask(x)
    skip_ref[...] = x
    z_ref[...] = _dot(x, w_ref[...])


def _tc5_body(s_ref, invc_ref, b_ref, w_ref, z_ref):
    x = jax.nn.relu((s_ref[0] + s_ref[1]) * invc_ref[...]
                    + b_ref[...][None, :])
    z_ref[...] = _rowmask(_dot(x, w_ref[...]))


def _tc_head_body(s_ref, invc_ref, b_ref, g_ref, bb_ref, skip2_ref,
                  skip4_ref, w1_ref, b1_ref, w2_ref, b2_ref, w3_ref, b3_ref,
                  wo_ref, bo_ref, y_ref):
    t = (s_ref[0] + s_ref[1]) * invc_ref[...] + b_ref[...][None, :]
    x = jax.nn.relu(_bn(t, g_ref[...], bb_ref[...]))
    x = x + skip2_ref[...] + skip4_ref[...]
    x = jax.nn.relu(_dot(x, w1_ref[...]) + b1_ref[...][None, :])
    x = jax.nn.relu(_dot(x, w2_ref[...]) + b2_ref[...][None, :])
    x = jax.nn.relu(_dot(x, w3_ref[...]) + b3_ref[...][None, :])
    x = _dot(x, wo_ref[...]) + bo_ref[...][None, :]
    y_ref[...] = jax.nn.sigmoid(x[:_N])


def _tc(body, out_shape):
    return pl.pallas_call(body, out_shape=out_shape)


_f32 = jnp.float32
_sh = jax.ShapeDtypeStruct
_ZPS = [_sh((_V, 80), _f32), _sh((_V, 16), _f32)]
_Z1 = _sh((_V, 16), _f32)


def kernel(x, edge_index, params):
    p = params
    pad = jnp.full((_EPAD - _EL,), _N, jnp.int32)
    loop = jnp.arange(_N, dtype=jnp.int32)
    src = jnp.concatenate([edge_index[0].astype(jnp.int32), loop, pad])
    src = src.reshape(_NW, _NB, _B)
    dst = jnp.concatenate([edge_index[1].astype(jnp.int32), loop, pad])
    dst = dst.reshape(_NW, _NB, _B)
    xpad = jnp.pad(x, ((0, _V - _N), (0, 0)))

    zps, pd = _tc(_tc0_body, _ZPS)(
        xpad, p["conv0_w"], p["conv0_u"], p["conv0_c"])
    s, d = _sc_feast4_deg(zps, pd, src, dst)

    invc, zps, pd = _tc(_tc1_body, [_Z1] + _ZPS)(
        s, d, p["conv0_b"], p["conv1_w"], p["conv1_u"], p["conv1_c"])
    (s,) = _sc_feast4(zps, pd, src, dst)

    skip2, zps, pd = _tc(_tc2_body, [_Z1] + _ZPS)(
        s, invc, p["conv1_b"], p["conv2_w"], p["conv2_u"], p["conv2_c"])
    (s,) = _sc_feast4(zps, pd, src, dst)

    z = _tc(_tc3_body, _Z1)(
        s, invc, p["conv2_b"], p["bn1_g"], p["bn1_b"], p["conv3_w"])
    (s,) = _sc_feast1(z, src, dst)

    skip4, z = _tc(_tc4_body, [_Z1, _Z1])(
        s, invc, p["conv3_b"], p["conv4_w"])
    (s,) = _sc_feast1(z, src, dst)

    z = _tc(_tc5_body, _Z1)(s, invc, p["conv4_b"], p["conv5_w"])
    (s,) = _sc_feast1(z, src, dst)

    y = _tc(_tc_head_body, _sh((_N, 1), _f32))(
        s, invc, p["conv5_b"], p["bn2_g"], p["bn2_b"], skip2, skip4,
        p["lin1_w"], p["lin1_b"], p["lin2_w"], p["lin2_b"],
        p["lin3_w"], p["lin3_b"], p["out_w"], p["out_b"])
    return y


# speedup vs baseline: 8.8431x; 1.0517x over previous
import jax
import jax.numpy as jnp
from jax import lax
from jax.experimental import pallas as pl
from jax.experimental.pallas import tpu as pltpu
from jax.experimental.pallas import tpu_sc as plsc

_N = 10000
_D = 128
_V = 10112
_E = 320000
_EL = _E + _N
_NW = 32
_B = 128
_NB = 82
_EPW = _NB * _B
_EPAD = _NW * _EPW
_RPT = _V // 16
_EPS = 1e-5


def _make_sc_layer(heads, with_deg, pipelined=True):
    zw = 16 * heads + (16 if heads > 1 else 0)
    mesh = plsc.VectorSubcoreMesh(core_axis_name="c", subcore_axis_name="s")
    n_out = 2 if with_deg else 1
    out_type = [jax.ShapeDtypeStruct((2, _V, 16), jnp.float32)] * n_out
    dma = pltpu.SemaphoreType.DMA
    scratch = [
        pltpu.VMEM((_NB + 1, _B), jnp.int32),
        pltpu.VMEM((_NB + 1, _B), jnp.int32),
        pltpu.VMEM((_B, zw), jnp.float32),
        pltpu.VMEM((_B, zw), jnp.float32),
        pltpu.VMEM((_RPT, 16), jnp.float32),
        pltpu.VMEM_SHARED((_V, 16), jnp.float32),
        dma, dma,
    ]
    if heads > 1:
        scratch += [
            pltpu.VMEM((_B, 16), jnp.float32),
            pltpu.VMEM((_B, 16), jnp.float32),
        ]
    if with_deg:
        scratch += [
            pltpu.VMEM((_B, 16), jnp.float32),
            pltpu.VMEM_SHARED((_V, 16), jnp.float32),
        ]

    def body(*refs):
        if heads > 1:
            z_hbm, pd_hbm, src_hbm, dst_hbm = refs[:4]
            k = 4
        else:
            z_hbm, src_hbm, dst_hbm = refs[:3]
            k = 3
        out_s = refs[k]; k += 1
        if with_deg:
            out_d = refs[k]; k += 1
        sidx, didx, z0, z1, zrow, acc, gz0, gz1 = refs[k:k + 8]
        k += 8
        zb = (z0, z1)
        gz = (gz0, gz1)
        if heads > 1:
            pdbuf, mbuf = refs[k:k + 2]
            k += 2
        if with_deg:
            ones, dacc = refs[k:k + 2]

        cid = lax.axis_index("c")
        sid = lax.axis_index("s")
        w = cid * 16 + sid

        zero16 = jnp.zeros((16,), jnp.float32)

        def zero_body(i, _):
            zrow[i, :] = zero16
            return 0
        lax.fori_loop(0, _RPT, zero_body, 0)
        pltpu.sync_copy(zrow, acc.at[pl.ds(sid * _RPT, _RPT)])
        if with_deg:
            pltpu.sync_copy(zrow, dacc.at[pl.ds(sid * _RPT, _RPT)])
            one16 = jnp.ones((16,), jnp.float32)

            def one_body(i, _):
                ones[i, :] = one16
                return 0
            lax.fori_loop(0, _B, one_body, 0)
        pltpu.sync_copy(src_hbm.at[w], sidx.at[pl.ds(0, _NB)])
        pltpu.sync_copy(dst_hbm.at[w], didx.at[pl.ds(0, _NB)])
        zero16i = jnp.zeros((16,), jnp.int32)
        for cc in range(_B // 16):
            sidx[_NB, pl.ds(16 * cc, 16)] = zero16i
        plsc.subcore_barrier()

        def gathers(b, slot):
            return [pltpu.make_async_copy(z_hbm.at[sidx.at[b]], zb[slot],
                                          gz[slot])]

        def step(b, slot, prefetch=True):
            if heads > 1:
                gathers(b, slot)[0].wait()
                if prefetch:
                    gathers(b + 1, 1 - slot)[0].start()
                pltpu.sync_copy(pd_hbm.at[didx.at[b]], pdbuf)
                zbuf = zb[slot]

                def edge(e):
                    t = zbuf[e, pl.ds(16 * heads, 16)] - pdbuf[e, :]
                    ev = jnp.exp(t)
                    es = [ev[h] for h in range(heads)]
                    s = es[0]
                    for h in range(1, heads):
                        s = s + es[h]
                    m = zbuf[e, 0:16] * es[0]
                    for h in range(1, heads):
                        m = m + zbuf[e, pl.ds(16 * h, 16)] * es[h]
                    mbuf[e, :] = m / jnp.broadcast_to(s, (16,))

                def egroup(g, _):
                    for j in range(16):
                        edge(g * 16 + j)
                    return 0
                lax.fori_loop(0, _B // 16, egroup, 0)
                pltpu.sync_copy(mbuf, acc.at[didx.at[b]], add=True)
            else:
                gathers(b, slot)[0].wait()
                if prefetch:
                    for d in gathers(b + 1, 1 - slot):
                        d.start()
                pltpu.sync_copy(zb[slot], acc.at[didx.at[b]], add=True)
            if with_deg:
                pltpu.sync_copy(ones, dacc.at[didx.at[b]], add=True)

        if pipelined:
            gathers(0, 0)[0].start()

            def pair_body(g, _):
                for slot in (0, 1):
                    step(2 * g + slot, slot)
                return 0
            lax.fori_loop(0, _NB // 2, pair_body, 0)

            gathers(_NB, 0)[0].wait()
        else:
            def batch_body(b, _):
                gathers(b, 0)[0].start()
                step(b, 0, prefetch=False)
                return 0
            lax.fori_loop(0, _NB, batch_body, 0)

        plsc.subcore_barrier()
        sl = pl.ds(sid * _RPT, _RPT)
        pltpu.sync_copy(acc.at[sl], zrow)
        pltpu.sync_copy(zrow, out_s.at[cid].at[sl])
        if with_deg:
            pltpu.sync_copy(dacc.at[sl], zrow)
            pltpu.sync_copy(zrow, out_d.at[cid].at[sl])

    return pl.kernel(body, out_type=out_type, mesh=mesh,
                     scratch_types=scratch,
                     compiler_params=pltpu.CompilerParams(
                         needs_layout_passes=False,
                         use_tc_tiling_on_sc=False))


_sc_feast4_deg = _make_sc_layer(4, True)
_sc_feast4 = _make_sc_layer(4, False)
_sc_feast1 = _make_sc_layer(1, False)


def _rowmask(a):
    rows = lax.broadcasted_iota(jnp.int32, a.shape, 0)
    return jnp.where(rows < _N, a, 0.0)


def _dot(a, b):
    return jnp.dot(a, b, preferred_element_type=jnp.float32)


def _proj(x, w_ref, u_ref, c_ref, zps_ref, pd_ref):
    z = _rowmask(_dot(x, w_ref[...]))
    u = u_ref[...]
    up = jnp.pad(u, ((0, 0), (0, 16 - u.shape[1])))
    ps = _dot(x, up)
    zps_ref[...] = jnp.concatenate([z, ps], axis=1)
    c = c_ref[...]
    cp = jnp.pad(c, (0, 16 - c.shape[0]))
    pd_ref[...] = ps - cp[None, :]


def _tc0_body(x_ref, w_ref, u_ref, c_ref, zps_ref, pd_ref):
    _proj(x_ref[...], w_ref, u_ref, c_ref, zps_ref, pd_ref)


def _tc1_body(s_ref, d_ref, b_ref, w_ref, u_ref, c_ref,
              invc_ref, zps_ref, pd_ref):
    invc = 1.0 / jnp.maximum(d_ref[0] + d_ref[1], 1.0)
    invc_ref[...] = invc
    x = jax.nn.relu((s_ref[0] + s_ref[1]) * invc + b_ref[...][None, :])
    x = _rowmask(x)
    _proj(x, w_ref, u_ref, c_ref, zps_ref, pd_ref)


def _tc2_body(s_ref, invc_ref, b_ref, w_ref, u_ref, c_ref,
              skip_ref, zps_ref, pd_ref):
    t = _rowmask((s_ref[0] + s_ref[1]) * invc_ref[...] + b_ref[...][None, :])
    skip_ref[...] = t
    x = jax.nn.relu(t)
    _proj(x, w_ref, u_ref, c_ref, zps_ref, pd_ref)


def _bn(t, g, b):
    tn = t[:_N]
    mu = jnp.mean(tn, axis=0)
    var = jnp.mean((tn - mu[None, :]) ** 2, axis=0)
    return (t - mu[None, :]) / jnp.sqrt(var + _EPS) * g[None, :] + b[None, :]


def _tc3_body(s_ref, invc_ref, b_ref, g_ref, bb_ref, w_ref, z_ref):
    t = (s_ref[0] + s_ref[1]) * invc_ref[...] + b_ref[...][None, :]
    x = jax.nn.relu(_bn(t, g_ref[...], bb_ref[...]))
    z_ref[...] = _rowmask(_dot(x, w_ref[...]))


def _tc4_body(s_ref, invc_ref, b_ref, w_ref, skip_ref, z_ref):
    x = jax.nn.relu((s_ref[0] + s_ref[1]) * invc_ref[...]
                    + b_ref[...][None, :])
    x = _rowmask(x)
    skip_ref[...] = x
    z_ref[...] = _dot(x, w_ref[...])


def _tc5_body(s_ref, invc_ref, b_ref, w_ref, z_ref):
    x = jax.nn.relu((s_ref[0] + s_ref[1]) * invc_ref[...]
                    + b_ref[...][None, :])
    z_ref[...] = _rowmask(_dot(x, w_ref[...]))


def _tc_head_body(s_ref, invc_ref, b_ref, g_ref, bb_ref, skip2_ref,
                  skip4_ref, w1_ref, b1_ref, w2_ref, b2_ref, w3_ref, b3_ref,
                  wo_ref, bo_ref, y_ref):
    t = (s_ref[0] + s_ref[1]) * invc_ref[...] + b_ref[...][None, :]
    x = jax.nn.relu(_bn(t, g_ref[...], bb_ref[...]))
    x = x + skip2_ref[...] + skip4_ref[...]
    x = jax.nn.relu(_dot(x, w1_ref[...]) + b1_ref[...][None, :])
    x = jax.nn.relu(_dot(x, w2_ref[...]) + b2_ref[...][None, :])
    x = jax.nn.relu(_dot(x, w3_ref[...]) + b3_ref[...][None, :])
    x = _dot(x, wo_ref[...]) + bo_ref[...][None, :]
    y_ref[...] = jax.nn.sigmoid(x[:_N])


def _tc(body, out_shape):
    return pl.pallas_call(body, out_shape=out_shape)


_f32 = jnp.float32
_sh = jax.ShapeDtypeStruct
_ZPS = [_sh((_V, 80), _f32), _sh((_V, 16), _f32)]
_Z1 = _sh((_V, 16), _f32)


def kernel(x, edge_index, params):
    p = params
    pad = jnp.full((_EPAD - _EL,), _N, jnp.int32)
    loop = jnp.arange(_N, dtype=jnp.int32)
    src = jnp.concatenate([edge_index[0].astype(jnp.int32), loop, pad])
    src = src.reshape(_NW, _NB, _B)
    dst = jnp.concatenate([edge_index[1].astype(jnp.int32), loop, pad])
    dst = dst.reshape(_NW, _NB, _B)
    xpad = jnp.pad(x, ((0, _V - _N), (0, 0)))

    zps, pd = _tc(_tc0_body, _ZPS)(
        xpad, p["conv0_w"], p["conv0_u"], p["conv0_c"])
    s, d = _sc_feast4_deg(zps, pd, src, dst)

    invc, zps, pd = _tc(_tc1_body, [_Z1] + _ZPS)(
        s, d, p["conv0_b"], p["conv1_w"], p["conv1_u"], p["conv1_c"])
    (s,) = _sc_feast4(zps, pd, src, dst)

    skip2, zps, pd = _tc(_tc2_body, [_Z1] + _ZPS)(
        s, invc, p["conv1_b"], p["conv2_w"], p["conv2_u"], p["conv2_c"])
    (s,) = _sc_feast4(zps, pd, src, dst)

    z = _tc(_tc3_body, _Z1)(
        s, invc, p["conv2_b"], p["bn1_g"], p["bn1_b"], p["conv3_w"])
    (s,) = _sc_feast1(z, src, dst)

    skip4, z = _tc(_tc4_body, [_Z1, _Z1])(
        s, invc, p["conv3_b"], p["conv4_w"])
    (s,) = _sc_feast1(z, src, dst)

    z = _tc(_tc5_body, _Z1)(s, invc, p["conv4_b"], p["conv5_w"])
    (s,) = _sc_feast1(z, src, dst)

    y = _tc(_tc_head_body, _sh((_N, 1), _f32))(
        s, invc, p["conv5_b"], p["bn2_g"], p["bn2_b"], skip2, skip4,
        p["lin1_w"], p["lin1_b"], p["lin2_w"], p["lin2_b"],
        p["lin3_w"], p["lin3_b"], p["out_w"], p["out_b"])
    return y
